# baseline (speedup 1.0000x reference)
import functools

import jax
import jax.numpy as jnp
from jax import lax
from jax.experimental import pallas as pl
from jax.experimental.pallas import tpu as pltpu
from jax.experimental.pallas import tpu_sc as plsc

N = 50000
E = 800000
NODE_DIM = 128
H = 64
HH = H // 2
ED = 16
NCONV = 3
PDIM = 128
NCRY = 500
CPAD = 512
APC = 100
EPS = 1e-5

NCORES = 2
NSUB = 16
NW = NCORES * NSUB

EB = 2000
CH = 1000
CHS = 400

_mesh = plsc.VectorSubcoreMesh(core_axis_name="c", subcore_axis_name="s")
_sc_params = pltpu.CompilerParams(use_tc_tiling_on_sc=False)


@functools.partial(
    pl.kernel,
    out_type=[
        jax.ShapeDtypeStruct((E, H), jnp.bfloat16),
        jax.ShapeDtypeStruct((E, H), jnp.bfloat16),
    ],
    mesh=_mesh,
    scratch_types=[
        pltpu.VMEM((CH,), jnp.int32),
        pltpu.VMEM((CH,), jnp.int32),
        pltpu.VMEM((CH, H), jnp.bfloat16),
        pltpu.VMEM((CH, H), jnp.bfloat16),
        pltpu.SemaphoreType.DMA,
        pltpu.SemaphoreType.DMA,
    ],
    compiler_params=_sc_params,
)
def _gather2(h_hbm, src_hbm, dst_hbm, hs_hbm, hd_hbm,
             idx_s, idx_d, buf_s, buf_d, sem_s, sem_d):
    wid = lax.axis_index("s") * NCORES + lax.axis_index("c")
    per_w = E // NW
    base = wid * per_w

    def body(k, carry):
        off = base + k * CH
        pltpu.sync_copy(src_hbm.at[pl.ds(off, CH)], idx_s)
        pltpu.sync_copy(dst_hbm.at[pl.ds(off, CH)], idx_d)
        cp_s = pltpu.async_copy(h_hbm.at[idx_s], buf_s, sem_s)
        cp_d = pltpu.async_copy(h_hbm.at[idx_d], buf_d, sem_d)
        cp_s.wait()
        cp_d.wait()
        pltpu.sync_copy(buf_s, hs_hbm.at[pl.ds(off, CH)])
        pltpu.sync_copy(buf_d, hd_hbm.at[pl.ds(off, CH)])
        return carry

    lax.fori_loop(0, per_w // CH, body, 0)


@functools.partial(
    pl.kernel,
    out_type=jax.ShapeDtypeStruct((N, H), jnp.float32),
    mesh=_mesh,
    scratch_types=[
        pltpu.VMEM_SHARED((N, HH), jnp.float32),
        pltpu.VMEM((CHS,), jnp.int32),
        pltpu.VMEM((CHS, HH), jnp.float32),
    ],
    compiler_params=_sc_params,
)
def _scatter_add(msg_hbm, src_hbm, h_hbm, hnew_hbm, agg_sh, idx_v, buf_v):
    c = lax.axis_index("c")
    s = lax.axis_index("s")
    rows_pt = N // NSUB
    chunks_pt = E // CHS // NSUB

    for cc in range(NCORES):
        @pl.when(c == cc)
        def _(cc=cc):
            pltpu.sync_copy(
                h_hbm.at[pl.ds(s * rows_pt, rows_pt), pl.ds(cc * HH, HH)],
                agg_sh.at[pl.ds(s * rows_pt, rows_pt)])
            plsc.subcore_barrier()

            def body(k, carry):
                off = (s * chunks_pt + k) * CHS
                pltpu.sync_copy(src_hbm.at[pl.ds(off, CHS)], idx_v)
                pltpu.sync_copy(
                    msg_hbm.at[pl.ds(off, CHS), pl.ds(cc * HH, HH)], buf_v)
                pltpu.sync_copy(buf_v, agg_sh.at[idx_v], add=True)
                return carry

            lax.fori_loop(0, chunks_pt, body, 0)
            plsc.subcore_barrier()
            pltpu.sync_copy(
                agg_sh.at[pl.ds(s * rows_pt, rows_pt)],
                hnew_hbm.at[pl.ds(s * rows_pt, rows_pt), pl.ds(cc * HH, HH)])


@functools.partial(
    pl.kernel,
    out_type=jax.ShapeDtypeStruct((CPAD * APC, H), jnp.float32),
    mesh=_mesh,
    scratch_types=[
        pltpu.VMEM((CPAD * APC // NW,), jnp.int32),
        pltpu.VMEM((CPAD * APC // NW, H), jnp.float32),
        pltpu.SemaphoreType.DMA,
    ],
    compiler_params=_sc_params,
)
def _pool_gather(h_hbm, idx_hbm, out_hbm, idx_v, buf_v, sem):
    wid = lax.axis_index("s") * NCORES + lax.axis_index("c")
    per_w = CPAD * APC // NW
    base = wid * per_w
    pltpu.sync_copy(idx_hbm.at[pl.ds(base, per_w)], idx_v)
    pltpu.async_copy(h_hbm.at[idx_v], buf_v, sem).wait()
    pltpu.sync_copy(buf_v, out_hbm.at[pl.ds(base, per_w)])


def _embed_body(nf, w, b, h_ref, h16_ref):
    h = (jnp.dot(nf[...], w[...], preferred_element_type=jnp.float32)
         + b[...])
    h_ref[...] = h
    h16_ref[...] = h.astype(jnp.bfloat16)


def _embed(node_feats, W_emb, b_emb):
    grid = N // EB
    return pl.pallas_call(
        _embed_body,
        grid=(grid,),
        in_specs=[
            pl.BlockSpec((EB, NODE_DIM), lambda i: (i, 0)),
            pl.BlockSpec((NODE_DIM, H), lambda i: (0, 0)),
            pl.BlockSpec((1, H), lambda i: (0, 0)),
        ],
        out_specs=[
            pl.BlockSpec((EB, H), lambda i: (i, 0)),
            pl.BlockSpec((EB, H), lambda i: (i, 0)),
        ],
        out_shape=[
            jax.ShapeDtypeStruct((N, H), jnp.float32),
            jax.ShapeDtypeStruct((N, H), jnp.bfloat16),
        ],
    )(node_feats, W_emb, b_emb)


def _to16_body(h, h16_ref):
    h16_ref[...] = h[...].astype(jnp.bfloat16)


def _to16(h):
    grid = N // EB
    return pl.pallas_call(
        _to16_body,
        grid=(grid,),
        in_specs=[pl.BlockSpec((EB, H), lambda i: (i, 0))],
        out_specs=pl.BlockSpec((EB, H), lambda i: (i, 0)),
        out_shape=jax.ShapeDtypeStruct((N, H), jnp.bfloat16),
    )(h)


def _zstats_body(hs, hd, ef, w1a, w2a, w3a, ba, w1b, w2b, w3b, bb,
                 za_ref, zb_ref, sa_ref, qa_ref, sb_ref, qb_ref):
    f32 = jnp.float32
    hsf = hs[...].astype(f32)
    hdf = hd[...].astype(f32)
    za = (jnp.dot(hsf, w1a[...], preferred_element_type=f32)
          + jnp.dot(hdf, w2a[...], preferred_element_type=f32)
          + jnp.dot(ef[...], w3a[...], preferred_element_type=f32)
          + ba[...])
    zb = (jnp.dot(hsf, w1b[...], preferred_element_type=f32)
          + jnp.dot(hdf, w2b[...], preferred_element_type=f32)
          + jnp.dot(ef[...], w3b[...], preferred_element_type=f32)
          + bb[...])
    za_ref[...] = za
    zb_ref[...] = zb

    @pl.when(pl.program_id(0) == 0)
    def _():
        sa_ref[...] = jnp.zeros_like(sa_ref)
        qa_ref[...] = jnp.zeros_like(qa_ref)
        sb_ref[...] = jnp.zeros_like(sb_ref)
        qb_ref[...] = jnp.zeros_like(qb_ref)

    sa_ref[...] += jnp.sum(za, axis=0, keepdims=True)
    qa_ref[...] += jnp.sum(za * za, axis=0, keepdims=True)
    sb_ref[...] += jnp.sum(zb, axis=0, keepdims=True)
    qb_ref[...] += jnp.sum(zb * zb, axis=0, keepdims=True)


def _zstats(hs, hd, ef, w1a, w2a, w3a, ba, w1b, w2b, w3b, bb):
    grid = E // EB
    row = lambda i: (i, 0)
    fix = lambda i: (0, 0)
    return pl.pallas_call(
        _zstats_body,
        grid=(grid,),
        in_specs=[
            pl.BlockSpec((EB, H), row),
            pl.BlockSpec((EB, H), row),
            pl.BlockSpec((EB, ED), row),
            pl.BlockSpec((H, H), fix),
            pl.BlockSpec((H, H), fix),
            pl.BlockSpec((ED, H), fix),
            pl.BlockSpec((1, H), fix),
            pl.BlockSpec((H, H), fix),
            pl.BlockSpec((H, H), fix),
            pl.BlockSpec((ED, H), fix),
            pl.BlockSpec((1, H), fix),
        ],
        out_specs=[
            pl.BlockSpec((EB, H), row),
            pl.BlockSpec((EB, H), row),
            pl.BlockSpec((1, H), fix),
            pl.BlockSpec((1, H), fix),
            pl.BlockSpec((1, H), fix),
            pl.BlockSpec((1, H), fix),
        ],
        out_shape=[
            jax.ShapeDtypeStruct((E, H), jnp.float32),
            jax.ShapeDtypeStruct((E, H), jnp.float32),
            jax.ShapeDtypeStruct((1, H), jnp.float32),
            jax.ShapeDtypeStruct((1, H), jnp.float32),
            jax.ShapeDtypeStruct((1, H), jnp.float32),
            jax.ShapeDtypeStruct((1, H), jnp.float32),
        ],
    )(hs, hd, ef, w1a, w2a, w3a, ba, w1b, w2b, w3b, bb)


def _msg_body(za, zb, sa, qa, sb, qb, ga, bta, gb, btb, msg_ref):
    inv_e = jnp.float32(1.0 / E)
    mua = sa[...] * inv_e
    vara = qa[...] * inv_e - mua * mua
    mub = sb[...] * inv_e
    varb = qb[...] * inv_e - mub * mub
    zna = (za[...] - mua) / jnp.sqrt(vara + EPS) * ga[...] + bta[...]
    znb = (zb[...] - mub) / jnp.sqrt(varb + EPS) * gb[...] + btb[...]
    msg_ref[...] = jax.nn.sigmoid(zna) * jax.nn.softplus(znb)


def _msg(za, zb, sa, qa, sb, qb, ga, bta, gb, btb):
    grid = E // EB
    row = lambda i: (i, 0)
    fix = lambda i: (0, 0)
    return pl.pallas_call(
        _msg_body,
        grid=(grid,),
        in_specs=[
            pl.BlockSpec((EB, H), row),
            pl.BlockSpec((EB, H), row),
        ] + [pl.BlockSpec((1, H), fix)] * 8,
        out_specs=pl.BlockSpec((EB, H), row),
        out_shape=jax.ShapeDtypeStruct((E, H), jnp.float32),
    )(za, zb, sa, qa, sb, qb, ga, bta, gb, btb)


_CB = 64
_RB = _CB * APC


def _head_body(rows, wfc, bfc, wout, bout, out_ref):
    f32 = jnp.float32
    ci = lax.broadcasted_iota(jnp.int32, (_CB, _RB), 0)
    ri = lax.broadcasted_iota(jnp.int32, (_CB, _RB), 1)
    sel = jnp.where(lax.div(ri, APC) == ci, f32(1.0 / APC), f32(0.0))
    pooled = jnp.dot(sel, rows[...], preferred_element_type=f32,
                     precision=lax.Precision.HIGHEST)
    fc = jax.nn.softplus(
        jnp.dot(pooled, wfc[...], preferred_element_type=f32) + bfc[...])
    out_ref[...] = jnp.dot(fc, wout[...], preferred_element_type=f32) + bout[...]


def _head(rows, W_fc, b_fc, W_out, b_out):
    grid = CPAD // _CB
    return pl.pallas_call(
        _head_body,
        grid=(grid,),
        in_specs=[
            pl.BlockSpec((_RB, H), lambda i: (i, 0)),
            pl.BlockSpec((H, PDIM), lambda i: (0, 0)),
            pl.BlockSpec((1, PDIM), lambda i: (0, 0)),
            pl.BlockSpec((PDIM, 1), lambda i: (0, 0)),
            pl.BlockSpec((1, 1), lambda i: (0, 0)),
        ],
        out_specs=pl.BlockSpec((_CB, 1), lambda i: (i, 0)),
        out_shape=jax.ShapeDtypeStruct((CPAD, 1), jnp.float32),
    )(rows, W_fc, b_fc, W_out, b_out)


def kernel(node_feats, edge_index, edge_feats, crystal_atom_idx,
           W_emb, b_emb, W_conv, b_conv, gamma, beta,
           W_fc, b_fc, W_out, b_out):
    src = edge_index[0]
    dst = edge_index[1]

    h, h16 = _embed(node_feats, W_emb, b_emb.reshape(1, H))

    for l in range(NCONV):
        Wl = W_conv[l]
        w1a, w1b = Wl[:H, :H], Wl[:H, H:]
        w2a, w2b = Wl[H:2 * H, :H], Wl[H:2 * H, H:]
        w3a, w3b = Wl[2 * H:, :H], Wl[2 * H:, H:]
        ba, bb = b_conv[l, :H].reshape(1, H), b_conv[l, H:].reshape(1, H)
        ga, gb = gamma[l, :H].reshape(1, H), gamma[l, H:].reshape(1, H)
        bta, btb = beta[l, :H].reshape(1, H), beta[l, H:].reshape(1, H)

        hs, hd = _gather2(h16, src, dst)
        za, zb, sa, qa, sb, qb = _zstats(
            hs, hd, edge_feats, w1a, w2a, w3a, ba, w1b, w2b, w3b, bb)
        msg = _msg(za, zb, sa, qa, sb, qb, ga, bta, gb, btb)
        h = _scatter_add(msg, src, h)
        if l + 1 < NCONV:
            h16 = _to16(h)

    cidx = jnp.pad(crystal_atom_idx, ((0, CPAD - NCRY), (0, 0))).reshape(-1)
    rows = _pool_gather(h, cidx)
    out = _head(rows, W_fc, b_fc.reshape(1, PDIM), W_out, b_out.reshape(1, 1))
    return out[:NCRY]


# speedup vs baseline: 1.0875x; 1.0875x over previous
import functools

import jax
import jax.numpy as jnp
from jax import lax
from jax.experimental import pallas as pl
from jax.experimental.pallas import tpu as pltpu
from jax.experimental.pallas import tpu_sc as plsc

N = 50000
E = 800000
NODE_DIM = 128
H = 64
HH = H // 2
ED = 16
NCONV = 3
PDIM = 128
NCRY = 500
CPAD = 512
APC = 100
EPS = 1e-5

NCORES = 2
NSUB = 16
NW = NCORES * NSUB

EB = 2000
CH = 1000
CHS = 400

_mesh = plsc.VectorSubcoreMesh(core_axis_name="c", subcore_axis_name="s")
_sc_params = pltpu.CompilerParams(use_tc_tiling_on_sc=False)


@functools.partial(
    pl.kernel,
    out_type=jax.ShapeDtypeStruct((E, 2 * H), jnp.bfloat16),
    mesh=_mesh,
    scratch_types=[
        pltpu.VMEM((CH,), jnp.int32),
        pltpu.VMEM((CH,), jnp.int32),
        pltpu.VMEM((CH, H), jnp.bfloat16),
        pltpu.VMEM((CH, H), jnp.bfloat16),
        pltpu.SemaphoreType.DMA,
        pltpu.SemaphoreType.DMA,
    ],
    compiler_params=_sc_params,
)
def _gather2(h_hbm, src_hbm, dst_hbm, hh_hbm,
             idx_s, idx_d, buf_s, buf_d, sem_s, sem_d):
    wid = lax.axis_index("s") * NCORES + lax.axis_index("c")
    per_w = E // NW
    base = wid * per_w

    def body(k, carry):
        off = base + k * CH
        pltpu.sync_copy(src_hbm.at[pl.ds(off, CH)], idx_s)
        pltpu.sync_copy(dst_hbm.at[pl.ds(off, CH)], idx_d)
        cp_s = pltpu.async_copy(h_hbm.at[idx_s], buf_s, sem_s)
        cp_d = pltpu.async_copy(h_hbm.at[idx_d], buf_d, sem_d)
        cp_s.wait()
        cp_d.wait()
        pltpu.sync_copy(buf_s, hh_hbm.at[pl.ds(off, CH), pl.ds(0, H)])
        pltpu.sync_copy(buf_d, hh_hbm.at[pl.ds(off, CH), pl.ds(H, H)])
        return carry

    lax.fori_loop(0, per_w // CH, body, 0)


@functools.partial(
    pl.kernel,
    out_type=jax.ShapeDtypeStruct((N, H), jnp.float32),
    mesh=_mesh,
    scratch_types=[
        pltpu.VMEM_SHARED((N, HH), jnp.float32),
        pltpu.VMEM((CHS,), jnp.int32),
        pltpu.VMEM((CHS, HH), jnp.float32),
    ],
    compiler_params=_sc_params,
)
def _scatter_add(msg_hbm, src_hbm, h_hbm, hnew_hbm, agg_sh, idx_v, buf_v):
    c = lax.axis_index("c")
    s = lax.axis_index("s")
    rows_pt = N // NSUB
    chunks_pt = E // CHS // NSUB

    for cc in range(NCORES):
        @pl.when(c == cc)
        def _(cc=cc):
            pltpu.sync_copy(
                h_hbm.at[pl.ds(s * rows_pt, rows_pt), pl.ds(cc * HH, HH)],
                agg_sh.at[pl.ds(s * rows_pt, rows_pt)])
            plsc.subcore_barrier()

            def body(k, carry):
                off = (s * chunks_pt + k) * CHS
                pltpu.sync_copy(src_hbm.at[pl.ds(off, CHS)], idx_v)
                pltpu.sync_copy(
                    msg_hbm.at[pl.ds(off, CHS), pl.ds(cc * HH, HH)], buf_v)
                pltpu.sync_copy(buf_v, agg_sh.at[idx_v], add=True)
                return carry

            lax.fori_loop(0, chunks_pt, body, 0)
            plsc.subcore_barrier()
            pltpu.sync_copy(
                agg_sh.at[pl.ds(s * rows_pt, rows_pt)],
                hnew_hbm.at[pl.ds(s * rows_pt, rows_pt), pl.ds(cc * HH, HH)])


@functools.partial(
    pl.kernel,
    out_type=jax.ShapeDtypeStruct((CPAD * APC, H), jnp.float32),
    mesh=_mesh,
    scratch_types=[
        pltpu.VMEM((CPAD * APC // NW,), jnp.int32),
        pltpu.VMEM((CPAD * APC // NW, H), jnp.float32),
        pltpu.SemaphoreType.DMA,
    ],
    compiler_params=_sc_params,
)
def _pool_gather(h_hbm, idx_hbm, out_hbm, idx_v, buf_v, sem):
    wid = lax.axis_index("s") * NCORES + lax.axis_index("c")
    per_w = CPAD * APC // NW
    base = wid * per_w
    pltpu.sync_copy(idx_hbm.at[pl.ds(base, per_w)], idx_v)
    pltpu.async_copy(h_hbm.at[idx_v], buf_v, sem).wait()
    pltpu.sync_copy(buf_v, out_hbm.at[pl.ds(base, per_w)])


def _embed_body(nf, w, b, h_ref, h16_ref):
    h = (jnp.dot(nf[...], w[...], preferred_element_type=jnp.float32)
         + b[...])
    h_ref[...] = h
    h16_ref[...] = h.astype(jnp.bfloat16)


def _embed(node_feats, W_emb, b_emb):
    grid = N // EB
    return pl.pallas_call(
        _embed_body,
        grid=(grid,),
        in_specs=[
            pl.BlockSpec((EB, NODE_DIM), lambda i: (i, 0)),
            pl.BlockSpec((NODE_DIM, H), lambda i: (0, 0)),
            pl.BlockSpec((1, H), lambda i: (0, 0)),
        ],
        out_specs=[
            pl.BlockSpec((EB, H), lambda i: (i, 0)),
            pl.BlockSpec((EB, H), lambda i: (i, 0)),
        ],
        out_shape=[
            jax.ShapeDtypeStruct((N, H), jnp.float32),
            jax.ShapeDtypeStruct((N, H), jnp.bfloat16),
        ],
    )(node_feats, W_emb, b_emb)


def _to16_body(h, h16_ref):
    h16_ref[...] = h[...].astype(jnp.bfloat16)


def _to16(h):
    grid = N // EB
    return pl.pallas_call(
        _to16_body,
        grid=(grid,),
        in_specs=[pl.BlockSpec((EB, H), lambda i: (i, 0))],
        out_specs=pl.BlockSpec((EB, H), lambda i: (i, 0)),
        out_shape=jax.ShapeDtypeStruct((N, H), jnp.bfloat16),
    )(h)


def _zstats_body(hh, ef, w12a, w3a, ba, w12b, w3b, bb,
                 za_ref, zb_ref, sa_ref, qa_ref, sb_ref, qb_ref):
    f32 = jnp.float32
    hhf = hh[...].astype(f32)
    za = (jnp.dot(hhf, w12a[...], preferred_element_type=f32)
          + jnp.dot(ef[...], w3a[...], preferred_element_type=f32)
          + ba[...])
    zb = (jnp.dot(hhf, w12b[...], preferred_element_type=f32)
          + jnp.dot(ef[...], w3b[...], preferred_element_type=f32)
          + bb[...])
    za_ref[...] = za
    zb_ref[...] = zb

    @pl.when(pl.program_id(0) == 0)
    def _():
        sa_ref[...] = jnp.zeros_like(sa_ref)
        qa_ref[...] = jnp.zeros_like(qa_ref)
        sb_ref[...] = jnp.zeros_like(sb_ref)
        qb_ref[...] = jnp.zeros_like(qb_ref)

    sa_ref[...] += jnp.sum(za, axis=0, keepdims=True)
    qa_ref[...] += jnp.sum(za * za, axis=0, keepdims=True)
    sb_ref[...] += jnp.sum(zb, axis=0, keepdims=True)
    qb_ref[...] += jnp.sum(zb * zb, axis=0, keepdims=True)


def _zstats(hh, ef, w12a, w3a, ba, w12b, w3b, bb):
    grid = E // EB
    row = lambda i: (i, 0)
    fix = lambda i: (0, 0)
    return pl.pallas_call(
        _zstats_body,
        grid=(grid,),
        in_specs=[
            pl.BlockSpec((EB, 2 * H), row),
            pl.BlockSpec((EB, ED), row),
            pl.BlockSpec((2 * H, H), fix),
            pl.BlockSpec((ED, H), fix),
            pl.BlockSpec((1, H), fix),
            pl.BlockSpec((2 * H, H), fix),
            pl.BlockSpec((ED, H), fix),
            pl.BlockSpec((1, H), fix),
        ],
        out_specs=[
            pl.BlockSpec((EB, H), row),
            pl.BlockSpec((EB, H), row),
            pl.BlockSpec((1, H), fix),
            pl.BlockSpec((1, H), fix),
            pl.BlockSpec((1, H), fix),
            pl.BlockSpec((1, H), fix),
        ],
        out_shape=[
            jax.ShapeDtypeStruct((E, H), jnp.float32),
            jax.ShapeDtypeStruct((E, H), jnp.float32),
            jax.ShapeDtypeStruct((1, H), jnp.float32),
            jax.ShapeDtypeStruct((1, H), jnp.float32),
            jax.ShapeDtypeStruct((1, H), jnp.float32),
            jax.ShapeDtypeStruct((1, H), jnp.float32),
        ],
    )(hh, ef, w12a, w3a, ba, w12b, w3b, bb)


def _msg_body(za, zb, sa, qa, sb, qb, ga, bta, gb, btb, msg_ref):
    inv_e = jnp.float32(1.0 / E)
    mua = sa[...] * inv_e
    vara = qa[...] * inv_e - mua * mua
    mub = sb[...] * inv_e
    varb = qb[...] * inv_e - mub * mub
    zna = (za[...] - mua) / jnp.sqrt(vara + EPS) * ga[...] + bta[...]
    znb = (zb[...] - mub) / jnp.sqrt(varb + EPS) * gb[...] + btb[...]
    msg_ref[...] = jax.nn.sigmoid(zna) * jax.nn.softplus(znb)


def _msg(za, zb, sa, qa, sb, qb, ga, bta, gb, btb):
    grid = E // EB
    row = lambda i: (i, 0)
    fix = lambda i: (0, 0)
    return pl.pallas_call(
        _msg_body,
        grid=(grid,),
        in_specs=[
            pl.BlockSpec((EB, H), row),
            pl.BlockSpec((EB, H), row),
        ] + [pl.BlockSpec((1, H), fix)] * 8,
        out_specs=pl.BlockSpec((EB, H), row),
        out_shape=jax.ShapeDtypeStruct((E, H), jnp.float32),
    )(za, zb, sa, qa, sb, qb, ga, bta, gb, btb)


_CB = 64
_RB = _CB * APC


def _head_body(rows, wfc, bfc, wout, bout, out_ref):
    f32 = jnp.float32
    ci = lax.broadcasted_iota(jnp.int32, (_CB, _RB), 0)
    ri = lax.broadcasted_iota(jnp.int32, (_CB, _RB), 1)
    sel = jnp.where(lax.div(ri, APC) == ci, f32(1.0 / APC), f32(0.0))
    pooled = jnp.dot(sel, rows[...], preferred_element_type=f32,
                     precision=lax.Precision.HIGHEST)
    fc = jax.nn.softplus(
        jnp.dot(pooled, wfc[...], preferred_element_type=f32) + bfc[...])
    out_ref[...] = jnp.dot(fc, wout[...], preferred_element_type=f32) + bout[...]


def _head(rows, W_fc, b_fc, W_out, b_out):
    grid = CPAD // _CB
    return pl.pallas_call(
        _head_body,
        grid=(grid,),
        in_specs=[
            pl.BlockSpec((_RB, H), lambda i: (i, 0)),
            pl.BlockSpec((H, PDIM), lambda i: (0, 0)),
            pl.BlockSpec((1, PDIM), lambda i: (0, 0)),
            pl.BlockSpec((PDIM, 1), lambda i: (0, 0)),
            pl.BlockSpec((1, 1), lambda i: (0, 0)),
        ],
        out_specs=pl.BlockSpec((_CB, 1), lambda i: (i, 0)),
        out_shape=jax.ShapeDtypeStruct((CPAD, 1), jnp.float32),
    )(rows, W_fc, b_fc, W_out, b_out)


def kernel(node_feats, edge_index, edge_feats, crystal_atom_idx,
           W_emb, b_emb, W_conv, b_conv, gamma, beta,
           W_fc, b_fc, W_out, b_out):
    src = edge_index[0]
    dst = edge_index[1]

    h, h16 = _embed(node_feats, W_emb, b_emb.reshape(1, H))

    for l in range(NCONV):
        Wl = W_conv[l]
        w12a, w12b = Wl[:2 * H, :H], Wl[:2 * H, H:]
        w3a, w3b = Wl[2 * H:, :H], Wl[2 * H:, H:]
        ba, bb = b_conv[l, :H].reshape(1, H), b_conv[l, H:].reshape(1, H)
        ga, gb = gamma[l, :H].reshape(1, H), gamma[l, H:].reshape(1, H)
        bta, btb = beta[l, :H].reshape(1, H), beta[l, H:].reshape(1, H)

        hh = _gather2(h16, src, dst)
        za, zb, sa, qa, sb, qb = _zstats(
            hh, edge_feats, w12a, w3a, ba, w12b, w3b, bb)
        msg = _msg(za, zb, sa, qa, sb, qb, ga, bta, gb, btb)
        h = _scatter_add(msg, src, h)
        if l + 1 < NCONV:
            h16 = _to16(h)

    cidx = jnp.pad(crystal_atom_idx, ((0, CPAD - NCRY), (0, 0))).reshape(-1)
    rows = _pool_gather(h, cidx)
    out = _head(rows, W_fc, b_fc.reshape(1, PDIM), W_out, b_out.reshape(1, 1))
    return out[:NCRY]


# speedup vs baseline: 1.1052x; 1.0162x over previous
import functools

import jax
import jax.numpy as jnp
from jax import lax
from jax.experimental import pallas as pl
from jax.experimental.pallas import tpu as pltpu
from jax.experimental.pallas import tpu_sc as plsc

N = 50000
E = 800000
NSPLIT = 2
EH = E // NSPLIT
NODE_DIM = 128
H = 64
HH = H // 2
ED = 16
NCONV = 3
PDIM = 128
NCRY = 500
CPAD = 512
APC = 100
EPS = 1e-5

NCORES = 2
NSUB = 16
NW = NCORES * NSUB

EB = 2000
CH = 1000
CHS = 400

_mesh = plsc.VectorSubcoreMesh(core_axis_name="c", subcore_axis_name="s")
_sc_params = pltpu.CompilerParams(use_tc_tiling_on_sc=False)


def _cdiv(a, b):
    return (a + b - 1) // b


@functools.cache
def _make_gather2(ne):
    nchunks = ne // CH
    iters = _cdiv(nchunks, NW)

    @functools.partial(
        pl.kernel,
        out_type=[
            jax.ShapeDtypeStruct((ne, H), jnp.float32),
            jax.ShapeDtypeStruct((ne, H), jnp.float32),
        ],
        mesh=_mesh,
        scratch_types=[
            pltpu.VMEM((CH,), jnp.int32),
            pltpu.VMEM((CH,), jnp.int32),
            pltpu.VMEM((CH, H), jnp.float32),
            pltpu.VMEM((CH, H), jnp.float32),
            pltpu.SemaphoreType.DMA,
            pltpu.SemaphoreType.DMA,
        ],
        compiler_params=_sc_params,
    )
    def _gather2(h_hbm, src_hbm, dst_hbm, hs_hbm, hd_hbm,
                 idx_s, idx_d, buf_s, buf_d, sem_s, sem_d):
        wid = lax.axis_index("s") * NCORES + lax.axis_index("c")

        def body(k, carry):
            cid = wid + k * NW

            @pl.when(cid < nchunks)
            def _():
                off = cid * CH
                pltpu.sync_copy(src_hbm.at[pl.ds(off, CH)], idx_s)
                pltpu.sync_copy(dst_hbm.at[pl.ds(off, CH)], idx_d)
                cp_s = pltpu.async_copy(h_hbm.at[idx_s], buf_s, sem_s)
                cp_d = pltpu.async_copy(h_hbm.at[idx_d], buf_d, sem_d)
                cp_s.wait()
                cp_d.wait()
                pltpu.sync_copy(buf_s, hs_hbm.at[pl.ds(off, CH)])
                pltpu.sync_copy(buf_d, hd_hbm.at[pl.ds(off, CH)])

            return carry

        lax.fori_loop(0, iters, body, 0)

    return _gather2


@functools.cache
def _make_scatter_add(ne):
    nchunks = ne // CHS
    iters = _cdiv(nchunks, NSUB)
    rows_pt = N // NSUB

    @functools.partial(
        pl.kernel,
        out_type=jax.ShapeDtypeStruct((N, H), jnp.float32),
        mesh=_mesh,
        scratch_types=[
            pltpu.VMEM_SHARED((N, HH), jnp.float32),
            pltpu.VMEM((CHS,), jnp.int32),
            pltpu.VMEM((CHS, HH), jnp.float32),
        ],
        compiler_params=_sc_params,
    )
    def _scatter_add(msg0_hbm, msg1_hbm, src0_hbm, src1_hbm, h_hbm, hnew_hbm,
                     agg_sh, idx_v, buf_v):
        c = lax.axis_index("c")
        s = lax.axis_index("s")

        for cc in range(NCORES):
            @pl.when(c == cc)
            def _(cc=cc):
                pltpu.sync_copy(
                    h_hbm.at[pl.ds(s * rows_pt, rows_pt), pl.ds(cc * HH, HH)],
                    agg_sh.at[pl.ds(s * rows_pt, rows_pt)])
                plsc.subcore_barrier()

                for msg_hbm, src_hbm in ((msg0_hbm, src0_hbm),
                                         (msg1_hbm, src1_hbm)):
                    def body(k, carry, msg_hbm=msg_hbm, src_hbm=src_hbm):
                        cid = s + k * NSUB

                        @pl.when(cid < nchunks)
                        def _():
                            off = cid * CHS
                            pltpu.sync_copy(src_hbm.at[pl.ds(off, CHS)], idx_v)
                            pltpu.sync_copy(
                                msg_hbm.at[pl.ds(off, CHS), pl.ds(cc * HH, HH)],
                                buf_v)
                            pltpu.sync_copy(buf_v, agg_sh.at[idx_v], add=True)

                        return carry

                    lax.fori_loop(0, iters, body, 0)

                plsc.subcore_barrier()
                pltpu.sync_copy(
                    agg_sh.at[pl.ds(s * rows_pt, rows_pt)],
                    hnew_hbm.at[pl.ds(s * rows_pt, rows_pt), pl.ds(cc * HH, HH)])

    return _scatter_add


@functools.partial(
    pl.kernel,
    out_type=jax.ShapeDtypeStruct((CPAD * APC, H), jnp.float32),
    mesh=_mesh,
    scratch_types=[
        pltpu.VMEM((CPAD * APC // NW,), jnp.int32),
        pltpu.VMEM((CPAD * APC // NW, H), jnp.float32),
        pltpu.SemaphoreType.DMA,
    ],
    compiler_params=_sc_params,
)
def _pool_gather(h_hbm, idx_hbm, out_hbm, idx_v, buf_v, sem):
    wid = lax.axis_index("s") * NCORES + lax.axis_index("c")
    per_w = CPAD * APC // NW
    base = wid * per_w
    pltpu.sync_copy(idx_hbm.at[pl.ds(base, per_w)], idx_v)
    pltpu.async_copy(h_hbm.at[idx_v], buf_v, sem).wait()
    pltpu.sync_copy(buf_v, out_hbm.at[pl.ds(base, per_w)])


def _embed_body(nf, w, b, h_ref):
    h_ref[...] = (jnp.dot(nf[...], w[...], preferred_element_type=jnp.float32)
                  + b[...])


def _embed(node_feats, W_emb, b_emb):
    grid = N // EB
    return pl.pallas_call(
        _embed_body,
        grid=(grid,),
        in_specs=[
            pl.BlockSpec((EB, NODE_DIM), lambda i: (i, 0)),
            pl.BlockSpec((NODE_DIM, H), lambda i: (0, 0)),
            pl.BlockSpec((1, H), lambda i: (0, 0)),
        ],
        out_specs=pl.BlockSpec((EB, H), lambda i: (i, 0)),
        out_shape=jax.ShapeDtypeStruct((N, H), jnp.float32),
    )(node_feats, W_emb, b_emb)


def _zstats_body(hs, hd, ef, w1a, w2a, w3a, ba, w1b, w2b, w3b, bb,
                 za_ref, zb_ref, sa_ref, qa_ref, sb_ref, qb_ref):
    f32 = jnp.float32
    za = (jnp.dot(hs[...], w1a[...], preferred_element_type=f32)
          + jnp.dot(hd[...], w2a[...], preferred_element_type=f32)
          + jnp.dot(ef[...], w3a[...], preferred_element_type=f32)
          + ba[...])
    zb = (jnp.dot(hs[...], w1b[...], preferred_element_type=f32)
          + jnp.dot(hd[...], w2b[...], preferred_element_type=f32)
          + jnp.dot(ef[...], w3b[...], preferred_element_type=f32)
          + bb[...])
    za_ref[...] = za
    zb_ref[...] = zb

    @pl.when(pl.program_id(0) == 0)
    def _():
        sa_ref[...] = jnp.zeros_like(sa_ref)
        qa_ref[...] = jnp.zeros_like(qa_ref)
        sb_ref[...] = jnp.zeros_like(sb_ref)
        qb_ref[...] = jnp.zeros_like(qb_ref)

    sa_ref[...] += jnp.sum(za, axis=0, keepdims=True)
    qa_ref[...] += jnp.sum(za * za, axis=0, keepdims=True)
    sb_ref[...] += jnp.sum(zb, axis=0, keepdims=True)
    qb_ref[...] += jnp.sum(zb * zb, axis=0, keepdims=True)


@functools.cache
def _make_zstats(ne):
    grid = ne // EB
    row = lambda i: (i, 0)
    fix = lambda i: (0, 0)
    return pl.pallas_call(
        _zstats_body,
        grid=(grid,),
        in_specs=[
            pl.BlockSpec((EB, H), row),
            pl.BlockSpec((EB, H), row),
            pl.BlockSpec((EB, ED), row),
            pl.BlockSpec((H, H), fix),
            pl.BlockSpec((H, H), fix),
            pl.BlockSpec((ED, H), fix),
            pl.BlockSpec((1, H), fix),
            pl.BlockSpec((H, H), fix),
            pl.BlockSpec((H, H), fix),
            pl.BlockSpec((ED, H), fix),
            pl.BlockSpec((1, H), fix),
        ],
        out_specs=[
            pl.BlockSpec((EB, H), row),
            pl.BlockSpec((EB, H), row),
            pl.BlockSpec((1, H), fix),
            pl.BlockSpec((1, H), fix),
            pl.BlockSpec((1, H), fix),
            pl.BlockSpec((1, H), fix),
        ],
        out_shape=[
            jax.ShapeDtypeStruct((ne, H), jnp.float32),
            jax.ShapeDtypeStruct((ne, H), jnp.float32),
            jax.ShapeDtypeStruct((1, H), jnp.float32),
            jax.ShapeDtypeStruct((1, H), jnp.float32),
            jax.ShapeDtypeStruct((1, H), jnp.float32),
            jax.ShapeDtypeStruct((1, H), jnp.float32),
        ],
    )


def _msg_body(za, zb, sa, qa, sb, qb, ga, bta, gb, btb, msg_ref):
    inv_e = jnp.float32(1.0 / E)
    mua = sa[...] * inv_e
    vara = qa[...] * inv_e - mua * mua
    mub = sb[...] * inv_e
    varb = qb[...] * inv_e - mub * mub
    zna = (za[...] - mua) / jnp.sqrt(vara + EPS) * ga[...] + bta[...]
    znb = (zb[...] - mub) / jnp.sqrt(varb + EPS) * gb[...] + btb[...]
    msg_ref[...] = jax.nn.sigmoid(zna) * jax.nn.softplus(znb)


@functools.cache
def _make_msg(ne):
    grid = ne // EB
    row = lambda i: (i, 0)
    fix = lambda i: (0, 0)
    return pl.pallas_call(
        _msg_body,
        grid=(grid,),
        in_specs=[
            pl.BlockSpec((EB, H), row),
            pl.BlockSpec((EB, H), row),
        ] + [pl.BlockSpec((1, H), fix)] * 8,
        out_specs=pl.BlockSpec((EB, H), row),
        out_shape=jax.ShapeDtypeStruct((ne, H), jnp.float32),
    )


_CB = 64
_RB = _CB * APC


def _head_body(rows, wfc, bfc, wout, bout, out_ref):
    f32 = jnp.float32
    ci = lax.broadcasted_iota(jnp.int32, (_CB, _RB), 0)
    ri = lax.broadcasted_iota(jnp.int32, (_CB, _RB), 1)
    sel = jnp.where(lax.div(ri, APC) == ci, f32(1.0 / APC), f32(0.0))
    pooled = jnp.dot(sel, rows[...], preferred_element_type=f32,
                     precision=lax.Precision.HIGHEST)
    fc = jax.nn.softplus(
        jnp.dot(pooled, wfc[...], preferred_element_type=f32) + bfc[...])
    out_ref[...] = jnp.dot(fc, wout[...], preferred_element_type=f32) + bout[...]


def _head(rows, W_fc, b_fc, W_out, b_out):
    grid = CPAD // _CB
    return pl.pallas_call(
        _head_body,
        grid=(grid,),
        in_specs=[
            pl.BlockSpec((_RB, H), lambda i: (i, 0)),
            pl.BlockSpec((H, PDIM), lambda i: (0, 0)),
            pl.BlockSpec((1, PDIM), lambda i: (0, 0)),
            pl.BlockSpec((PDIM, 1), lambda i: (0, 0)),
            pl.BlockSpec((1, 1), lambda i: (0, 0)),
        ],
        out_specs=pl.BlockSpec((_CB, 1), lambda i: (i, 0)),
        out_shape=jax.ShapeDtypeStruct((CPAD, 1), jnp.float32),
    )(rows, W_fc, b_fc, W_out, b_out)


def kernel(node_feats, edge_index, edge_feats, crystal_atom_idx,
           W_emb, b_emb, W_conv, b_conv, gamma, beta,
           W_fc, b_fc, W_out, b_out):
    srcs = [edge_index[0, i * EH:(i + 1) * EH] for i in range(NSPLIT)]
    dsts = [edge_index[1, i * EH:(i + 1) * EH] for i in range(NSPLIT)]
    efs = [edge_feats[i * EH:(i + 1) * EH] for i in range(NSPLIT)]

    gather2 = _make_gather2(EH)
    zstats = _make_zstats(EH)
    msgk = _make_msg(EH)
    scat = _make_scatter_add(EH)

    h = _embed(node_feats, W_emb, b_emb.reshape(1, H))

    for l in range(NCONV):
        Wl = W_conv[l]
        w1a, w1b = Wl[:H, :H], Wl[:H, H:]
        w2a, w2b = Wl[H:2 * H, :H], Wl[H:2 * H, H:]
        w3a, w3b = Wl[2 * H:, :H], Wl[2 * H:, H:]
        ba, bb = b_conv[l, :H].reshape(1, H), b_conv[l, H:].reshape(1, H)
        ga, gb = gamma[l, :H].reshape(1, H), gamma[l, H:].reshape(1, H)
        bta, btb = beta[l, :H].reshape(1, H), beta[l, H:].reshape(1, H)

        gath = [gather2(h, srcs[i], dsts[i]) for i in range(NSPLIT)]
        zs = [zstats(gath[i][0], gath[i][1], efs[i],
                     w1a, w2a, w3a, ba, w1b, w2b, w3b, bb)
              for i in range(NSPLIT)]
        sa = sum(z[2] for z in zs)
        qa = sum(z[3] for z in zs)
        sb = sum(z[4] for z in zs)
        qb = sum(z[5] for z in zs)
        msgs = [msgk(zs[i][0], zs[i][1], sa, qa, sb, qb, ga, bta, gb, btb)
                for i in range(NSPLIT)]
        h = scat(msgs[0], msgs[1], srcs[0], srcs[1], h)

    cidx = jnp.pad(crystal_atom_idx, ((0, CPAD - NCRY), (0, 0))).reshape(-1)
    rows = _pool_gather(h, cidx)
    out = _head(rows, W_fc, b_fc.reshape(1, PDIM), W_out, b_out.reshape(1, 1))
    return out[:NCRY]


# speedup vs baseline: 1.1845x; 1.0718x over previous
import functools

import jax
import jax.numpy as jnp
from jax import lax
from jax.experimental import pallas as pl
from jax.experimental.pallas import tpu as pltpu
from jax.experimental.pallas import tpu_sc as plsc

N = 50000
E = 800000
NSPLIT = 4
EH = E // NSPLIT
NODE_DIM = 128
H = 64
HH = H // 2
ED = 16
NCONV = 3
PDIM = 128
NCRY = 500
CPAD = 512
APC = 100
EPS = 1e-5

NCORES = 2
NSUB = 16
NW = NCORES * NSUB

EB = 2000
CH = 1000
CHS = 400

_mesh = plsc.VectorSubcoreMesh(core_axis_name="c", subcore_axis_name="s")
_sc_params = pltpu.CompilerParams(use_tc_tiling_on_sc=False)


def _cdiv(a, b):
    return (a + b - 1) // b


@functools.cache
def _make_gather2(ne):
    nchunks = ne // CH
    iters = _cdiv(nchunks, NW)

    @functools.partial(
        pl.kernel,
        out_type=[
            jax.ShapeDtypeStruct((ne, H), jnp.float32),
            jax.ShapeDtypeStruct((ne, H), jnp.float32),
        ],
        mesh=_mesh,
        scratch_types=[
            pltpu.VMEM((CH,), jnp.int32),
            pltpu.VMEM((CH,), jnp.int32),
            pltpu.VMEM((CH, H), jnp.float32),
            pltpu.VMEM((CH, H), jnp.float32),
            pltpu.SemaphoreType.DMA,
            pltpu.SemaphoreType.DMA,
        ],
        compiler_params=_sc_params,
    )
    def _gather2(h_hbm, src_hbm, dst_hbm, hs_hbm, hd_hbm,
                 idx_s, idx_d, buf_s, buf_d, sem_s, sem_d):
        wid = lax.axis_index("s") * NCORES + lax.axis_index("c")

        def body(k, carry):
            cid = wid + k * NW

            @pl.when(cid < nchunks)
            def _():
                off = cid * CH
                pltpu.sync_copy(src_hbm.at[pl.ds(off, CH)], idx_s)
                pltpu.sync_copy(dst_hbm.at[pl.ds(off, CH)], idx_d)
                cp_s = pltpu.async_copy(h_hbm.at[idx_s], buf_s, sem_s)
                cp_d = pltpu.async_copy(h_hbm.at[idx_d], buf_d, sem_d)
                cp_s.wait()
                cp_d.wait()
                pltpu.sync_copy(buf_s, hs_hbm.at[pl.ds(off, CH)])
                pltpu.sync_copy(buf_d, hd_hbm.at[pl.ds(off, CH)])

            return carry

        lax.fori_loop(0, iters, body, 0)

    return _gather2


@functools.cache
def _make_scatter_add(ne):
    nchunks = ne // CHS
    iters = _cdiv(nchunks, NSUB)
    rows_pt = N // NSUB

    @functools.partial(
        pl.kernel,
        out_type=jax.ShapeDtypeStruct((N, H), jnp.float32),
        mesh=_mesh,
        scratch_types=[
            pltpu.VMEM_SHARED((N, HH), jnp.float32),
            pltpu.VMEM((CHS,), jnp.int32),
            pltpu.VMEM((CHS, HH), jnp.float32),
        ],
        compiler_params=_sc_params,
    )
    def _scatter_add(msg_hbm, src_hbm, h_hbm, hnew_hbm, agg_sh, idx_v, buf_v):
        c = lax.axis_index("c")
        s = lax.axis_index("s")

        for cc in range(NCORES):
            @pl.when(c == cc)
            def _(cc=cc):
                pltpu.sync_copy(
                    h_hbm.at[pl.ds(s * rows_pt, rows_pt), pl.ds(cc * HH, HH)],
                    agg_sh.at[pl.ds(s * rows_pt, rows_pt)])
                plsc.subcore_barrier()

                def body(k, carry):
                    cid = s + k * NSUB

                    @pl.when(cid < nchunks)
                    def _():
                        off = cid * CHS
                        pltpu.sync_copy(src_hbm.at[pl.ds(off, CHS)], idx_v)
                        pltpu.sync_copy(
                            msg_hbm.at[pl.ds(off, CHS), pl.ds(cc * HH, HH)],
                            buf_v)
                        pltpu.sync_copy(buf_v, agg_sh.at[idx_v], add=True)

                    return carry

                lax.fori_loop(0, iters, body, 0)

                plsc.subcore_barrier()
                pltpu.sync_copy(
                    agg_sh.at[pl.ds(s * rows_pt, rows_pt)],
                    hnew_hbm.at[pl.ds(s * rows_pt, rows_pt), pl.ds(cc * HH, HH)])

    return _scatter_add


@functools.partial(
    pl.kernel,
    out_type=jax.ShapeDtypeStruct((CPAD * APC, H), jnp.float32),
    mesh=_mesh,
    scratch_types=[
        pltpu.VMEM((CPAD * APC // NW,), jnp.int32),
        pltpu.VMEM((CPAD * APC // NW, H), jnp.float32),
        pltpu.SemaphoreType.DMA,
    ],
    compiler_params=_sc_params,
)
def _pool_gather(h_hbm, idx_hbm, out_hbm, idx_v, buf_v, sem):
    wid = lax.axis_index("s") * NCORES + lax.axis_index("c")
    per_w = CPAD * APC // NW
    base = wid * per_w
    pltpu.sync_copy(idx_hbm.at[pl.ds(base, per_w)], idx_v)
    pltpu.async_copy(h_hbm.at[idx_v], buf_v, sem).wait()
    pltpu.sync_copy(buf_v, out_hbm.at[pl.ds(base, per_w)])


def _embed_body(nf, w, b, h_ref):
    h_ref[...] = (jnp.dot(nf[...], w[...], preferred_element_type=jnp.float32)
                  + b[...])


def _embed(node_feats, W_emb, b_emb):
    grid = N // EB
    return pl.pallas_call(
        _embed_body,
        grid=(grid,),
        in_specs=[
            pl.BlockSpec((EB, NODE_DIM), lambda i: (i, 0)),
            pl.BlockSpec((NODE_DIM, H), lambda i: (0, 0)),
            pl.BlockSpec((1, H), lambda i: (0, 0)),
        ],
        out_specs=pl.BlockSpec((EB, H), lambda i: (i, 0)),
        out_shape=jax.ShapeDtypeStruct((N, H), jnp.float32),
    )(node_feats, W_emb, b_emb)


def _zstats_body(hs, hd, ef, w1a, w2a, w3a, ba, w1b, w2b, w3b, bb,
                 za_ref, zb_ref, sa_ref, qa_ref, sb_ref, qb_ref):
    f32 = jnp.float32
    za = (jnp.dot(hs[...], w1a[...], preferred_element_type=f32)
          + jnp.dot(hd[...], w2a[...], preferred_element_type=f32)
          + jnp.dot(ef[...], w3a[...], preferred_element_type=f32)
          + ba[...])
    zb = (jnp.dot(hs[...], w1b[...], preferred_element_type=f32)
          + jnp.dot(hd[...], w2b[...], preferred_element_type=f32)
          + jnp.dot(ef[...], w3b[...], preferred_element_type=f32)
          + bb[...])
    za_ref[...] = za
    zb_ref[...] = zb

    @pl.when(pl.program_id(0) == 0)
    def _():
        sa_ref[...] = jnp.zeros_like(sa_ref)
        qa_ref[...] = jnp.zeros_like(qa_ref)
        sb_ref[...] = jnp.zeros_like(sb_ref)
        qb_ref[...] = jnp.zeros_like(qb_ref)

    sa_ref[...] += jnp.sum(za, axis=0, keepdims=True)
    qa_ref[...] += jnp.sum(za * za, axis=0, keepdims=True)
    sb_ref[...] += jnp.sum(zb, axis=0, keepdims=True)
    qb_ref[...] += jnp.sum(zb * zb, axis=0, keepdims=True)


@functools.cache
def _make_zstats(ne):
    grid = ne // EB
    row = lambda i: (i, 0)
    fix = lambda i: (0, 0)
    return pl.pallas_call(
        _zstats_body,
        grid=(grid,),
        in_specs=[
            pl.BlockSpec((EB, H), row),
            pl.BlockSpec((EB, H), row),
            pl.BlockSpec((EB, ED), row),
            pl.BlockSpec((H, H), fix),
            pl.BlockSpec((H, H), fix),
            pl.BlockSpec((ED, H), fix),
            pl.BlockSpec((1, H), fix),
            pl.BlockSpec((H, H), fix),
            pl.BlockSpec((H, H), fix),
            pl.BlockSpec((ED, H), fix),
            pl.BlockSpec((1, H), fix),
        ],
        out_specs=[
            pl.BlockSpec((EB, H), row),
            pl.BlockSpec((EB, H), row),
            pl.BlockSpec((1, H), fix),
            pl.BlockSpec((1, H), fix),
            pl.BlockSpec((1, H), fix),
            pl.BlockSpec((1, H), fix),
        ],
        out_shape=[
            jax.ShapeDtypeStruct((ne, H), jnp.float32),
            jax.ShapeDtypeStruct((ne, H), jnp.float32),
            jax.ShapeDtypeStruct((1, H), jnp.float32),
            jax.ShapeDtypeStruct((1, H), jnp.float32),
            jax.ShapeDtypeStruct((1, H), jnp.float32),
            jax.ShapeDtypeStruct((1, H), jnp.float32),
        ],
    )


def _msg_body(za, zb, sa, qa, sb, qb, ga, bta, gb, btb, msg_ref):
    inv_e = jnp.float32(1.0 / E)
    mua = sa[...] * inv_e
    vara = qa[...] * inv_e - mua * mua
    mub = sb[...] * inv_e
    varb = qb[...] * inv_e - mub * mub
    zna = (za[...] - mua) / jnp.sqrt(vara + EPS) * ga[...] + bta[...]
    znb = (zb[...] - mub) / jnp.sqrt(varb + EPS) * gb[...] + btb[...]
    msg_ref[...] = jax.nn.sigmoid(zna) * jax.nn.softplus(znb)


@functools.cache
def _make_msg(ne):
    grid = ne // EB
    row = lambda i: (i, 0)
    fix = lambda i: (0, 0)
    return pl.pallas_call(
        _msg_body,
        grid=(grid,),
        in_specs=[
            pl.BlockSpec((EB, H), row),
            pl.BlockSpec((EB, H), row),
        ] + [pl.BlockSpec((1, H), fix)] * 8,
        out_specs=pl.BlockSpec((EB, H), row),
        out_shape=jax.ShapeDtypeStruct((ne, H), jnp.float32),
    )


_CB = 64
_RB = _CB * APC


def _head_body(rows, wfc, bfc, wout, bout, out_ref):
    f32 = jnp.float32
    ci = lax.broadcasted_iota(jnp.int32, (_CB, _RB), 0)
    ri = lax.broadcasted_iota(jnp.int32, (_CB, _RB), 1)
    sel = jnp.where(lax.div(ri, APC) == ci, f32(1.0 / APC), f32(0.0))
    pooled = jnp.dot(sel, rows[...], preferred_element_type=f32,
                     precision=lax.Precision.HIGHEST)
    fc = jax.nn.softplus(
        jnp.dot(pooled, wfc[...], preferred_element_type=f32) + bfc[...])
    out_ref[...] = jnp.dot(fc, wout[...], preferred_element_type=f32) + bout[...]


def _head(rows, W_fc, b_fc, W_out, b_out):
    grid = CPAD // _CB
    return pl.pallas_call(
        _head_body,
        grid=(grid,),
        in_specs=[
            pl.BlockSpec((_RB, H), lambda i: (i, 0)),
            pl.BlockSpec((H, PDIM), lambda i: (0, 0)),
            pl.BlockSpec((1, PDIM), lambda i: (0, 0)),
            pl.BlockSpec((PDIM, 1), lambda i: (0, 0)),
            pl.BlockSpec((1, 1), lambda i: (0, 0)),
        ],
        out_specs=pl.BlockSpec((_CB, 1), lambda i: (i, 0)),
        out_shape=jax.ShapeDtypeStruct((CPAD, 1), jnp.float32),
    )(rows, W_fc, b_fc, W_out, b_out)


def kernel(node_feats, edge_index, edge_feats, crystal_atom_idx,
           W_emb, b_emb, W_conv, b_conv, gamma, beta,
           W_fc, b_fc, W_out, b_out):
    srcs = [edge_index[0, i * EH:(i + 1) * EH] for i in range(NSPLIT)]
    dsts = [edge_index[1, i * EH:(i + 1) * EH] for i in range(NSPLIT)]
    efs = [edge_feats[i * EH:(i + 1) * EH] for i in range(NSPLIT)]

    gather2 = _make_gather2(EH)
    zstats = _make_zstats(EH)
    msgk = _make_msg(EH)
    scat = _make_scatter_add(EH)

    h = _embed(node_feats, W_emb, b_emb.reshape(1, H))

    for l in range(NCONV):
        Wl = W_conv[l]
        w1a, w1b = Wl[:H, :H], Wl[:H, H:]
        w2a, w2b = Wl[H:2 * H, :H], Wl[H:2 * H, H:]
        w3a, w3b = Wl[2 * H:, :H], Wl[2 * H:, H:]
        ba, bb = b_conv[l, :H].reshape(1, H), b_conv[l, H:].reshape(1, H)
        ga, gb = gamma[l, :H].reshape(1, H), gamma[l, H:].reshape(1, H)
        bta, btb = beta[l, :H].reshape(1, H), beta[l, H:].reshape(1, H)

        gath = [gather2(h, srcs[i], dsts[i]) for i in range(NSPLIT)]
        zs = [zstats(gath[i][0], gath[i][1], efs[i],
                     w1a, w2a, w3a, ba, w1b, w2b, w3b, bb)
              for i in range(NSPLIT)]
        sa = sum(z[2] for z in zs)
        qa = sum(z[3] for z in zs)
        sb = sum(z[4] for z in zs)
        qb = sum(z[5] for z in zs)
        msgs = [msgk(zs[i][0], zs[i][1], sa, qa, sb, qb, ga, bta, gb, btb)
                for i in range(NSPLIT)]
        for i in range(NSPLIT):
            h = scat(msgs[i], srcs[i], h)

    cidx = jnp.pad(crystal_atom_idx, ((0, CPAD - NCRY), (0, 0))).reshape(-1)
    rows = _pool_gather(h, cidx)
    out = _head(rows, W_fc, b_fc.reshape(1, PDIM), W_out, b_out.reshape(1, 1))
    return out[:NCRY]
